# baseline (speedup 1.0000x reference)
import functools

import jax
import jax.numpy as jnp
from jax import lax
from jax.experimental import pallas as pl
from jax.experimental.pallas import tpu as pltpu
from jax.experimental.pallas import tpu_sc as plsc

B = 16384
D = 64
N = 1000000

_info = plsc.get_sparse_core_info()
_NC, _NS, _L = _info.num_cores, _info.num_subcores, _info.num_lanes
_NW = _NC * _NS
_BPW = B // _NW


def _detile_body(*refs):
    uT_ref, iT_ref = refs[0], refs[1]
    outs = refs[2:2 + 2 * D]
    sem = refs[-1]
    copies = []
    for d in range(D):
        copies.append(pltpu.make_async_copy(uT_ref.at[d], outs[d], sem))
        copies.append(pltpu.make_async_copy(iT_ref.at[d], outs[D + d], sem))
    for c in copies:
        c.start()
    for c in copies:
        c.wait()


_detile = pl.pallas_call(
    _detile_body,
    out_shape=tuple(
        jax.ShapeDtypeStruct((N,), jnp.float32) for _ in range(2 * D)
    ),
    in_specs=[
        pl.BlockSpec(memory_space=pltpu.HBM),
        pl.BlockSpec(memory_space=pltpu.HBM),
    ],
    out_specs=tuple(
        pl.BlockSpec(memory_space=pltpu.HBM) for _ in range(2 * D)
    ),
    scratch_shapes=[pltpu.SemaphoreType.DMA],
)


def _mf_body(*refs):
    uid_hbm, iid_hbm = refs[0], refs[1]
    u_feats = refs[2:2 + D]
    i_feats = refs[2 + D:2 + 2 * D]
    ub_hbm, ib_hbm, gb_hbm = refs[2 + 2 * D], refs[3 + 2 * D], refs[4 + 2 * D]
    out_hbm = refs[5 + 2 * D]
    uidx_v, iidx_v, u_buf, i_buf, ub_v, ib_v, out_v, gb_v, sem = refs[6 + 2 * D:]

    wid = lax.axis_index("s") * _NC + lax.axis_index("c")
    base = wid * _BPW

    pltpu.sync_copy(uid_hbm.at[pl.ds(base, _BPW)], uidx_v)
    pltpu.sync_copy(iid_hbm.at[pl.ds(base, _BPW)], iidx_v)
    pltpu.sync_copy(gb_hbm, gb_v)

    copies = [
        pltpu.async_copy(ub_hbm.at[uidx_v], ub_v, sem),
        pltpu.async_copy(ib_hbm.at[iidx_v], ib_v, sem),
    ]
    for d in range(D):
        copies.append(pltpu.async_copy(u_feats[d].at[uidx_v], u_buf.at[d], sem))
        copies.append(pltpu.async_copy(i_feats[d].at[iidx_v], i_buf.at[d], sem))
    for c in copies:
        c.wait()

    gbv = gb_v[...]

    def group(g, carry):
        r0 = g * _L
        acc = ub_v[pl.ds(r0, _L)] + ib_v[pl.ds(r0, _L)] + gbv
        for d in range(D):
            acc = acc + u_buf[d, pl.ds(r0, _L)] * i_buf[d, pl.ds(r0, _L)]
        out_v[pl.ds(r0, _L)] = acc
        return carry

    lax.fori_loop(0, _BPW // _L, group, 0)
    pltpu.sync_copy(out_v, out_hbm.at[pl.ds(base, _BPW)])


_mf_sc = functools.partial(
    pl.kernel,
    out_type=jax.ShapeDtypeStruct((B,), jnp.float32),
    mesh=plsc.VectorSubcoreMesh(core_axis_name="c", subcore_axis_name="s"),
    compiler_params=pltpu.CompilerParams(needs_layout_passes=False, use_tc_tiling_on_sc=False),
    scratch_types=[
        pltpu.VMEM((_BPW,), jnp.int32),
        pltpu.VMEM((_BPW,), jnp.int32),
        pltpu.VMEM((D, _BPW), jnp.float32),
        pltpu.VMEM((D, _BPW), jnp.float32),
        pltpu.VMEM((_BPW,), jnp.float32),
        pltpu.VMEM((_BPW,), jnp.float32),
        pltpu.VMEM((_BPW,), jnp.float32),
        pltpu.VMEM((_L,), jnp.float32),
        pltpu.SemaphoreType.DMA,
    ],
)(_mf_body)


def kernel(user_ids, item_ids, user_emb, item_emb, user_bias, item_bias, global_bias):
    uid = user_ids.astype(jnp.int32)
    iid = item_ids.astype(jnp.int32)
    feats = _detile(user_emb.T, item_emb.T)
    ub = user_bias.reshape(-1)
    ib = item_bias.reshape(-1)
    gb = jnp.broadcast_to(global_bias.reshape(()), (_L,))
    return _mf_sc(uid, iid, *feats, ub, ib, gb)


# speedup vs baseline: 5.5108x; 5.5108x over previous
import functools

import jax
import jax.numpy as jnp
from jax import lax
from jax.experimental import pallas as pl
from jax.experimental.pallas import tpu as pltpu
from jax.experimental.pallas import tpu_sc as plsc

B = 16384
D = 64

_info = plsc.get_sparse_core_info()
_NC, _NS, _L = _info.num_cores, _info.num_subcores, _info.num_lanes
_NW = _NC * _NS
_BPW = B // _NW
_CHUNK = 128
_NCHUNK = _BPW // _CHUNK


def _mf_body(uid_hbm, iid_hbm, uemb_hbm, iemb_hbm, ub_hbm, ib_hbm, gb_hbm,
             out_hbm,
             uidx_v, iidx_v, urows_v, irows_v, ub_v, ib_v, out_v, gb_v, sem):
    wid = lax.axis_index("s") * _NC + lax.axis_index("c")
    base = wid * _BPW

    for j in range(_NCHUNK):
        pltpu.sync_copy(uid_hbm.at[pl.ds(base + j * _CHUNK, _CHUNK)], uidx_v.at[j])
        pltpu.sync_copy(iid_hbm.at[pl.ds(base + j * _CHUNK, _CHUNK)], iidx_v.at[j])
    pltpu.sync_copy(gb_hbm, gb_v)

    copies = []
    for j in range(_NCHUNK):
        sl = pl.ds(j * _CHUNK, _CHUNK)
        copies.append(pltpu.async_copy(uemb_hbm.at[uidx_v.at[j]], urows_v.at[sl], sem))
        copies.append(pltpu.async_copy(iemb_hbm.at[iidx_v.at[j]], irows_v.at[sl], sem))
        copies.append(pltpu.async_copy(ub_hbm.at[uidx_v.at[j]], ub_v.at[sl], sem))
        copies.append(pltpu.async_copy(ib_hbm.at[iidx_v.at[j]], ib_v.at[sl], sem))
    for c in copies:
        c.wait()

    gbv = gb_v[...]
    iota = lax.broadcasted_iota(jnp.int32, (_L,), 0)

    zcol = jnp.zeros((_L,), jnp.int32)

    def group(g, carry):
        r0 = g * _L
        rows = r0 + iota
        acc = plsc.load_gather(ub_v, [rows, zcol]) + plsc.load_gather(ib_v, [rows, zcol]) + gbv
        for l in range(_L):
            r = r0 + l
            p = urows_v[r, pl.ds(0, _L)] * irows_v[r, pl.ds(0, _L)]
            for k in range(1, D // _L):
                p = p + urows_v[r, pl.ds(k * _L, _L)] * irows_v[r, pl.ds(k * _L, _L)]
            s = jnp.sum(p)
            acc = jnp.where(iota == l, acc + s, acc)
        out_v[pl.ds(r0, _L)] = acc
        return carry

    lax.fori_loop(0, _BPW // _L, group, 0)
    pltpu.sync_copy(out_v, out_hbm.at[pl.ds(base, _BPW)])


_mf_sc = functools.partial(
    pl.kernel,
    out_type=jax.ShapeDtypeStruct((B,), jnp.float32),
    mesh=plsc.VectorSubcoreMesh(core_axis_name="c", subcore_axis_name="s"),
    compiler_params=pltpu.CompilerParams(needs_layout_passes=False, use_tc_tiling_on_sc=False),
    scratch_types=[
        pltpu.VMEM((_NCHUNK, _CHUNK), jnp.int32),
        pltpu.VMEM((_NCHUNK, _CHUNK), jnp.int32),
        pltpu.VMEM((_BPW, D), jnp.float32),
        pltpu.VMEM((_BPW, D), jnp.float32),
        pltpu.VMEM((_BPW, 1), jnp.float32),
        pltpu.VMEM((_BPW, 1), jnp.float32),
        pltpu.VMEM((_BPW,), jnp.float32),
        pltpu.VMEM((_L,), jnp.float32),
        pltpu.SemaphoreType.DMA,
    ],
)(_mf_body)


def kernel(user_ids, item_ids, user_emb, item_emb, user_bias, item_bias, global_bias):
    uid = user_ids.astype(jnp.int32)
    iid = item_ids.astype(jnp.int32)
    gb = jnp.broadcast_to(global_bias.reshape(()), (_L,))
    return _mf_sc(uid, iid, user_emb, item_emb, user_bias, item_bias, gb)


# speedup vs baseline: 13.9499x; 2.5314x over previous
import functools

import jax
import jax.numpy as jnp
from jax import lax
from jax.experimental import pallas as pl
from jax.experimental.pallas import tpu as pltpu
from jax.experimental.pallas import tpu_sc as plsc

B = 16384
D = 64

_info = plsc.get_sparse_core_info()
_NC, _NS, _L = _info.num_cores, _info.num_subcores, _info.num_lanes
_NW = _NC * _NS
_BPW = B // _NW
_CHUNK = 128
_NCHUNK = _BPW // _CHUNK


def _mf_body(uid_hbm, iid_hbm, uemb_hbm, iemb_hbm, ub_hbm, ib_hbm, gb_hbm,
             out_hbm,
             uidx_v, iidx_v, urows_v, irows_v, ub_v, ib_v, out_v, gb_v, sem):
    wid = lax.axis_index("s") * _NC + lax.axis_index("c")
    base = wid * _BPW

    for j in range(_NCHUNK):
        pltpu.sync_copy(uid_hbm.at[pl.ds(base + j * _CHUNK, _CHUNK)], uidx_v.at[j])
        pltpu.sync_copy(iid_hbm.at[pl.ds(base + j * _CHUNK, _CHUNK)], iidx_v.at[j])
    pltpu.sync_copy(gb_hbm, gb_v)

    copies = []
    for j in range(_NCHUNK):
        sl = pl.ds(j * _CHUNK, _CHUNK)
        copies.append(pltpu.async_copy(uemb_hbm.at[uidx_v.at[j]], urows_v.at[sl], sem))
        copies.append(pltpu.async_copy(iemb_hbm.at[iidx_v.at[j]], irows_v.at[sl], sem))
        copies.append(pltpu.async_copy(ub_hbm.at[uidx_v.at[j]], ub_v.at[sl], sem))
        copies.append(pltpu.async_copy(ib_hbm.at[iidx_v.at[j]], ib_v.at[sl], sem))
    for c in copies:
        c.wait()

    gbv = gb_v[...]
    iota = lax.broadcasted_iota(jnp.int32, (_L,), 0)

    def group(g, carry):
        r0 = g * _L
        acc = ub_v[pl.ds(r0, _L)] + ib_v[pl.ds(r0, _L)] + gbv
        for l in range(_L):
            r = r0 + l
            p = urows_v[r, pl.ds(0, _L)] * irows_v[r, pl.ds(0, _L)]
            for k in range(1, D // _L):
                p = p + urows_v[r, pl.ds(k * _L, _L)] * irows_v[r, pl.ds(k * _L, _L)]
            s = jnp.sum(p)
            acc = jnp.where(iota == l, acc + s, acc)
        out_v[pl.ds(r0, _L)] = acc
        return carry

    lax.fori_loop(0, _BPW // _L, group, 0)
    pltpu.sync_copy(out_v, out_hbm.at[pl.ds(base, _BPW)])


_mf_sc = functools.partial(
    pl.kernel,
    out_type=jax.ShapeDtypeStruct((B,), jnp.float32),
    mesh=plsc.VectorSubcoreMesh(core_axis_name="c", subcore_axis_name="s"),
    compiler_params=pltpu.CompilerParams(needs_layout_passes=False, use_tc_tiling_on_sc=False),
    scratch_types=[
        pltpu.VMEM((_NCHUNK, _CHUNK), jnp.int32),
        pltpu.VMEM((_NCHUNK, _CHUNK), jnp.int32),
        pltpu.VMEM((_BPW, D), jnp.float32),
        pltpu.VMEM((_BPW, D), jnp.float32),
        pltpu.VMEM((_BPW,), jnp.float32),
        pltpu.VMEM((_BPW,), jnp.float32),
        pltpu.VMEM((_BPW,), jnp.float32),
        pltpu.VMEM((_L,), jnp.float32),
        pltpu.SemaphoreType.DMA,
    ],
)(_mf_body)


def kernel(user_ids, item_ids, user_emb, item_emb, user_bias, item_bias, global_bias):
    uid = user_ids.astype(jnp.int32)
    iid = item_ids.astype(jnp.int32)
    ub = jnp.sum(user_bias, axis=1)
    ib = jnp.sum(item_bias, axis=1)
    gb = jnp.broadcast_to(global_bias.reshape(()), (_L,))
    return _mf_sc(uid, iid, user_emb, item_emb, ub, ib, gb)


# speedup vs baseline: 13.9730x; 1.0017x over previous
import functools

import jax
import jax.numpy as jnp
from jax import lax
from jax.experimental import pallas as pl
from jax.experimental.pallas import tpu as pltpu
from jax.experimental.pallas import tpu_sc as plsc

B = 16384
D = 64

_info = plsc.get_sparse_core_info()
_NC, _NS, _L = _info.num_cores, _info.num_subcores, _info.num_lanes
_NW = _NC * _NS
_BPW = B // _NW
_CHUNK = 128
_NCHUNK = _BPW // _CHUNK


def _mf_body(uid_hbm, iid_hbm, uemb_hbm, iemb_hbm, ub_hbm, ib_hbm, gb_hbm,
             out_hbm,
             uidx_v, iidx_v, urows_v, irows_v, ub_v, ib_v, out_v, gb_v, sem):
    wid = lax.axis_index("s") * _NC + lax.axis_index("c")
    base = wid * _BPW

    for j in range(_NCHUNK):
        pltpu.sync_copy(uid_hbm.at[pl.ds(base + j * _CHUNK, _CHUNK)], uidx_v.at[j])
        pltpu.sync_copy(iid_hbm.at[pl.ds(base + j * _CHUNK, _CHUNK)], iidx_v.at[j])
    pltpu.sync_copy(gb_hbm, gb_v)

    copies = []
    for j in range(_NCHUNK):
        sl = pl.ds(j * _CHUNK, _CHUNK)
        copies.append(pltpu.async_copy(uemb_hbm.at[uidx_v.at[j]], urows_v.at[sl], sem))
        copies.append(pltpu.async_copy(iemb_hbm.at[iidx_v.at[j]], irows_v.at[sl], sem))
        copies.append(pltpu.async_copy(ub_hbm.at[uidx_v.at[j]], ub_v.at[sl], sem))
        copies.append(pltpu.async_copy(ib_hbm.at[iidx_v.at[j]], ib_v.at[sl], sem))
    for c in copies:
        c.wait()

    gbv = gb_v[...]
    iota = lax.broadcasted_iota(jnp.int32, (_L,), 0)

    def group(g, carry):
        r0 = g * _L
        acc = ub_v[pl.ds(r0, _L)] + ib_v[pl.ds(r0, _L)] + gbv
        for l in range(_L):
            r = r0 + l
            p = urows_v[r, pl.ds(0, _L)] * irows_v[r, pl.ds(0, _L)]
            for k in range(1, D // _L):
                p = p + urows_v[r, pl.ds(k * _L, _L)] * irows_v[r, pl.ds(k * _L, _L)]
            s = jnp.sum(p)
            acc = jnp.where(iota == l, acc + s, acc)
        out_v[pl.ds(r0, _L)] = acc
        return carry

    lax.fori_loop(0, _BPW // _L, group, 0)
    pltpu.sync_copy(out_v, out_hbm.at[pl.ds(base, _BPW)])


_mf_sc = functools.partial(
    pl.kernel,
    out_type=jax.ShapeDtypeStruct((B,), jnp.float32),
    mesh=plsc.VectorSubcoreMesh(core_axis_name="c", subcore_axis_name="s"),
    compiler_params=pltpu.CompilerParams(needs_layout_passes=False, use_tc_tiling_on_sc=False),
    scratch_types=[
        pltpu.VMEM((_NCHUNK, _CHUNK), jnp.int32),
        pltpu.VMEM((_NCHUNK, _CHUNK), jnp.int32),
        pltpu.VMEM((_BPW, D), jnp.float32),
        pltpu.VMEM((_BPW, D), jnp.float32),
        pltpu.VMEM((_BPW,), jnp.float32),
        pltpu.VMEM((_BPW,), jnp.float32),
        pltpu.VMEM((_BPW,), jnp.float32),
        pltpu.VMEM((_L,), jnp.float32),
        pltpu.SemaphoreType.DMA,
    ],
)(_mf_body)


def kernel(user_ids, item_ids, user_emb, item_emb, user_bias, item_bias, global_bias):
    uid = user_ids.astype(jnp.int32)
    iid = item_ids.astype(jnp.int32)
    ub = user_bias @ jnp.ones((1,), jnp.float32)
    ib = item_bias @ jnp.ones((1,), jnp.float32)
    gb = jnp.broadcast_to(global_bias.reshape(()), (_L,))
    return _mf_sc(uid, iid, user_emb, item_emb, ub, ib, gb)


# speedup vs baseline: 13.9741x; 1.0001x over previous
import functools

import jax
import jax.numpy as jnp
from jax import lax
from jax.experimental import pallas as pl
from jax.experimental.pallas import tpu as pltpu
from jax.experimental.pallas import tpu_sc as plsc

B = 16384
D = 64

_info = plsc.get_sparse_core_info()
_NC, _NS, _L = _info.num_cores, _info.num_subcores, _info.num_lanes
_NW = _NC * _NS
_BPW = B // _NW
_CHUNK = 128
_NCHUNK = _BPW // _CHUNK


def _mf_body(uid_hbm, iid_hbm, uemb_hbm, iemb_hbm, bias_hbm, gb_hbm,
             out_hbm,
             uidx_v, iidx_v, urows_v, irows_v, ub_v, ib_v, out_v, gb_v, sem):
    wid = lax.axis_index("s") * _NC + lax.axis_index("c")
    base = wid * _BPW

    for j in range(_NCHUNK):
        pltpu.sync_copy(uid_hbm.at[pl.ds(base + j * _CHUNK, _CHUNK)], uidx_v.at[j])
        pltpu.sync_copy(iid_hbm.at[pl.ds(base + j * _CHUNK, _CHUNK)], iidx_v.at[j])
    pltpu.sync_copy(gb_hbm, gb_v)

    copies = []
    for j in range(_NCHUNK):
        sl = pl.ds(j * _CHUNK, _CHUNK)
        copies.append(pltpu.async_copy(uemb_hbm.at[uidx_v.at[j]], urows_v.at[sl], sem))
        copies.append(pltpu.async_copy(iemb_hbm.at[iidx_v.at[j]], irows_v.at[sl], sem))
        copies.append(pltpu.async_copy(bias_hbm.at[pl.ds(0, 1000000)].at[uidx_v.at[j]], ub_v.at[sl], sem))
        copies.append(pltpu.async_copy(bias_hbm.at[pl.ds(1000000, 1000000)].at[iidx_v.at[j]], ib_v.at[sl], sem))
    for c in copies:
        c.wait()

    gbv = gb_v[...]
    iota = lax.broadcasted_iota(jnp.int32, (_L,), 0)

    def group(g, carry):
        r0 = g * _L
        acc = ub_v[pl.ds(r0, _L)] + ib_v[pl.ds(r0, _L)] + gbv
        for l in range(_L):
            r = r0 + l
            p = urows_v[r, pl.ds(0, _L)] * irows_v[r, pl.ds(0, _L)]
            for k in range(1, D // _L):
                p = p + urows_v[r, pl.ds(k * _L, _L)] * irows_v[r, pl.ds(k * _L, _L)]
            s = jnp.sum(p)
            acc = jnp.where(iota == l, acc + s, acc)
        out_v[pl.ds(r0, _L)] = acc
        return carry

    lax.fori_loop(0, _BPW // _L, group, 0)
    pltpu.sync_copy(out_v, out_hbm.at[pl.ds(base, _BPW)])


_mf_sc = functools.partial(
    pl.kernel,
    out_type=jax.ShapeDtypeStruct((B,), jnp.float32),
    mesh=plsc.VectorSubcoreMesh(core_axis_name="c", subcore_axis_name="s"),
    compiler_params=pltpu.CompilerParams(needs_layout_passes=False, use_tc_tiling_on_sc=False),
    scratch_types=[
        pltpu.VMEM((_NCHUNK, _CHUNK), jnp.int32),
        pltpu.VMEM((_NCHUNK, _CHUNK), jnp.int32),
        pltpu.VMEM((_BPW, D), jnp.float32),
        pltpu.VMEM((_BPW, D), jnp.float32),
        pltpu.VMEM((_BPW,), jnp.float32),
        pltpu.VMEM((_BPW,), jnp.float32),
        pltpu.VMEM((_BPW,), jnp.float32),
        pltpu.VMEM((_L,), jnp.float32),
        pltpu.SemaphoreType.DMA,
    ],
)(_mf_body)


def kernel(user_ids, item_ids, user_emb, item_emb, user_bias, item_bias, global_bias):
    uid = user_ids.astype(jnp.int32)
    iid = item_ids.astype(jnp.int32)
    biases = jnp.concatenate([user_bias, item_bias], axis=0).reshape(-1)
    gb = jnp.broadcast_to(global_bias.reshape(()), (_L,))
    return _mf_sc(uid, iid, user_emb, item_emb, biases, gb)
